# baseline (speedup 1.0000x reference)
import jax
import jax.numpy as jnp
from jax import lax
from jax.experimental import pallas as pl
from jax.experimental.pallas import tpu as pltpu
from jax.experimental.pallas import tpu_sc as plsc

B = 4096
L = 50
D = 64
N = B * L
NC = 2
NS = 16
NW = NC * NS
N_W = N // NW
SUB = 128
CHUNK = 640
NSUB = CHUNK // SUB
NCHUNK = N_W // CHUNK
GROUPS = CHUNK // 16
IDX_ROWS = N_W // SUB


def _sc_body(uidx_hbm, midx_hbm, utab_hbm, mtab_hbm, out_hbm,
             uidx_v, midx_v, urows_v, mrows_v, out_v, sem_u, sem_m):
    wid = lax.axis_index("s") * NC + lax.axis_index("c")

    pltpu.sync_copy(uidx_hbm.at[wid], uidx_v)
    pltpu.sync_copy(midx_hbm.at[wid], midx_v)

    iota = lax.iota(jnp.int32, 16)

    def chunk_body(j, carry):
        copies = []
        for k in range(NSUB):
            copies.append(pltpu.async_copy(
                utab_hbm.at[uidx_v.at[j * NSUB + k]],
                urows_v.at[pl.ds(k * SUB, SUB)], sem_u))
            copies.append(pltpu.async_copy(
                mtab_hbm.at[midx_v.at[j * NSUB + k]],
                mrows_v.at[pl.ds(k * SUB, SUB)], sem_m))
        for cp in copies:
            cp.wait()

        def group_body(g, carry2):
            row = jnp.full((16,), g * 16, jnp.int32) + iota
            acc = jnp.zeros((16,), jnp.float32)
            for d in range(D):
                col = jnp.full((16,), d, jnp.int32)
                ug = plsc.load_gather(urows_v, [row, col])
                mg = plsc.load_gather(mrows_v, [row, col])
                acc = acc + ug * mg
            out_v[pl.ds(j * CHUNK + g * 16, 16)] = acc
            return carry2

        lax.fori_loop(0, GROUPS, group_body, 0)
        return carry

    lax.fori_loop(0, NCHUNK, chunk_body, 0)

    pltpu.sync_copy(out_v, out_hbm.at[pl.ds(wid * N_W, N_W)])


def kernel(user_id, movie_title, user_table, movie_table):
    uidx = user_id.reshape(NW, IDX_ROWS, SUB)
    midx = movie_title.reshape(NW, IDX_ROWS, SUB)
    mesh = plsc.VectorSubcoreMesh(core_axis_name="c", subcore_axis_name="s")
    out = pl.kernel(
        _sc_body,
        out_type=jax.ShapeDtypeStruct((N,), jnp.float32),
        mesh=mesh,
        scratch_types=[
            pltpu.VMEM((IDX_ROWS, SUB), jnp.int32),
            pltpu.VMEM((IDX_ROWS, SUB), jnp.int32),
            pltpu.VMEM((CHUNK, D), jnp.float32),
            pltpu.VMEM((CHUNK, D), jnp.float32),
            pltpu.VMEM((N_W,), jnp.float32),
            pltpu.SemaphoreType.DMA,
            pltpu.SemaphoreType.DMA,
        ],
        compiler_params=pltpu.CompilerParams(
            needs_layout_passes=False, use_tc_tiling_on_sc=False,
            disable_bounds_checks=True),
    )(uidx, midx, user_table, movie_table)
    return out.reshape(B, L)


# speedup vs baseline: 1.0500x; 1.0500x over previous
import jax
import jax.numpy as jnp
from jax import lax
from jax.experimental import pallas as pl
from jax.experimental.pallas import tpu as pltpu
from jax.experimental.pallas import tpu_sc as plsc

B = 4096
L = 50
D = 64
N = B * L
NC = 2
NS = 16
NW = NC * NS
N_W = N // NW
SUB = 80
CHUNK = 320
NSUB = CHUNK // SUB
NCHUNK = N_W // CHUNK
GROUPS = CHUNK // 16
IDX_ROWS = N_W // SUB


def _sc_body(uidx_hbm, midx_hbm, utab_hbm, mtab_hbm, out_hbm,
             uidx_v, midx_v, urows, mrows, out_v, sems):
    wid = lax.axis_index("s") * NC + lax.axis_index("c")

    pltpu.sync_copy(uidx_hbm.at[wid], uidx_v)
    pltpu.sync_copy(midx_hbm.at[wid], midx_v)

    iota = lax.iota(jnp.int32, 16)

    def issue(j, b):
        for k in range(NSUB):
            pltpu.async_copy(
                utab_hbm.at[uidx_v.at[j * NSUB + k]],
                urows.at[b].at[pl.ds(k * SUB, SUB)], sems.at[b, 0])
            pltpu.async_copy(
                mtab_hbm.at[midx_v.at[j * NSUB + k]],
                mrows.at[b].at[pl.ds(k * SUB, SUB)], sems.at[b, 1])

    def wait(j, b):
        for k in range(NSUB):
            pltpu.make_async_copy(
                utab_hbm.at[uidx_v.at[j * NSUB + k]],
                urows.at[b].at[pl.ds(k * SUB, SUB)], sems.at[b, 0]).wait()
            pltpu.make_async_copy(
                mtab_hbm.at[midx_v.at[j * NSUB + k]],
                mrows.at[b].at[pl.ds(k * SUB, SUB)], sems.at[b, 1]).wait()

    zero16 = jnp.zeros((16,), jnp.int32)

    def compute(j, b):
        ub = urows.at[b]
        mb = mrows.at[b]

        def group_body(g, carry):
            base = (g * 16 + iota) * D
            accs = [jnp.zeros((16,), jnp.float32) for _ in range(4)]
            for d in range(D):
                idx = base + d
                ug = plsc.load_gather(ub, [zero16, idx])
                mg = plsc.load_gather(mb, [zero16, idx])
                accs[d % 4] = accs[d % 4] + ug * mg
            acc = (accs[0] + accs[1]) + (accs[2] + accs[3])
            out_v[pl.ds(j * CHUNK + g * 16, 16)] = acc
            return carry

        lax.fori_loop(0, GROUPS, group_body, 0)

    issue(0, 0)
    issue(1, 1)

    def outer_body(i, carry):
        for b in range(2):
            j = 2 * i + b
            wait(j, b)
            compute(j, b)

            @pl.when(j + 2 < NCHUNK)
            def _():
                issue(j + 2, b)
        return carry

    lax.fori_loop(0, NCHUNK // 2, outer_body, 0, unroll=False)

    pltpu.sync_copy(out_v, out_hbm.at[pl.ds(wid * N_W, N_W)])


def kernel(user_id, movie_title, user_table, movie_table):
    uidx = user_id.reshape(NW, IDX_ROWS, SUB)
    midx = movie_title.reshape(NW, IDX_ROWS, SUB)
    mesh = plsc.VectorSubcoreMesh(core_axis_name="c", subcore_axis_name="s")
    out = pl.kernel(
        _sc_body,
        out_type=jax.ShapeDtypeStruct((N,), jnp.float32),
        mesh=mesh,
        scratch_types=[
            pltpu.VMEM((IDX_ROWS, SUB), jnp.int32),
            pltpu.VMEM((IDX_ROWS, SUB), jnp.int32),
            pltpu.VMEM((2, CHUNK, D), jnp.float32),
            pltpu.VMEM((2, CHUNK, D), jnp.float32),
            pltpu.VMEM((N_W,), jnp.float32),
            pltpu.SemaphoreType.DMA((2, 2)),
        ],
        compiler_params=pltpu.CompilerParams(
            needs_layout_passes=False, use_tc_tiling_on_sc=False,
            disable_bounds_checks=True),
    )(uidx, midx, user_table, movie_table)
    return out.reshape(B, L)
